# baseline (speedup 1.0000x reference)
import jax
import jax.numpy as jnp
from jax.experimental import pallas as pl
from jax.experimental.pallas import tpu as pltpu


def _round_up(x, m):
    return ((x + m - 1) // m) * m


def _make_body(n_rows, chunk):
    n_chunks = n_rows // chunk

    def _body(x_hbm, w_ref, b_ref, adj_hbm, o_hbm,
              xv, abuf, obuf, x_sem, in_sems, out_sems):
        pltpu.make_async_copy(adj_hbm.at[pl.ds(0, chunk), :],
                              abuf.at[0], in_sems.at[0]).start()
        pltpu.make_async_copy(x_hbm, xv, x_sem).start()
        if n_chunks > 1:
            pltpu.make_async_copy(adj_hbm.at[pl.ds(chunk, chunk), :],
                                  abuf.at[1], in_sems.at[1]).start()
        pltpu.make_async_copy(x_hbm, xv, x_sem).wait()

        for i in range(n_chunks):
            if i + 2 < n_chunks:
                s2 = (i + 2) % 3
                pltpu.make_async_copy(
                    adj_hbm.at[pl.ds((i + 2) * chunk, chunk), :],
                    abuf.at[s2], in_sems.at[s2]).start()
            sl = i % 3
            pltpu.make_async_copy(abuf.at[sl], abuf.at[sl],
                                  in_sems.at[sl]).wait()
            t = jnp.dot(abuf[sl], xv[...],
                        preferred_element_type=jnp.float32)
            ob = i % 2
            if i >= 2:
                pltpu.make_async_copy(obuf.at[ob], obuf.at[ob],
                                      out_sems.at[ob]).wait()
            obuf[ob] = jnp.dot(t, w_ref[...],
                               preferred_element_type=jnp.float32) + b_ref[...]
            pltpu.make_async_copy(obuf.at[ob],
                                  o_hbm.at[pl.ds(i * chunk, chunk), :],
                                  out_sems.at[ob]).start()

        for ob in range(min(2, n_chunks)):
            pltpu.make_async_copy(obuf.at[ob], obuf.at[ob],
                                  out_sems.at[ob]).wait()

    return _body


def kernel(x, adj, weight, bias):
    n, f_in = x.shape
    f_out = weight.shape[1]

    f_in_p = _round_up(f_in, 128)
    f_out_p = _round_up(f_out, 128)

    chunk = 256
    n_p = _round_up(n, chunk)

    x = x.astype(jnp.float32)
    if (n_p, f_in_p) != (n, f_in):
        x = jnp.pad(x, ((0, n_p - n), (0, f_in_p - f_in)))
    w = weight.astype(jnp.float32)
    if (f_in_p, f_out_p) != (f_in, f_out):
        w = jnp.pad(w, ((0, f_in_p - f_in), (0, f_out_p - f_out)))
    adj_p = adj if n_p == n else jnp.pad(adj, ((0, n_p - n), (0, n_p - n)))
    if bias is None:
        b = jnp.zeros((1, f_out_p), jnp.float32)
    else:
        b = jnp.pad(bias.reshape(1, f_out).astype(jnp.float32),
                    ((0, 0), (0, f_out_p - f_out)))

    out_p = pl.pallas_call(
        _make_body(n_p, chunk),
        out_shape=jax.ShapeDtypeStruct((n_p, f_out_p), jnp.float32),
        in_specs=[
            pl.BlockSpec(memory_space=pl.ANY),
            pl.BlockSpec((f_in_p, f_out_p), lambda: (0, 0)),
            pl.BlockSpec((1, f_out_p), lambda: (0, 0)),
            pl.BlockSpec(memory_space=pl.ANY),
        ],
        out_specs=pl.BlockSpec(memory_space=pl.ANY),
        scratch_shapes=[
            pltpu.VMEM((n_p, f_in_p), jnp.float32),
            pltpu.VMEM((3, chunk, n_p), jnp.float32),
            pltpu.VMEM((2, chunk, f_out_p), jnp.float32),
            pltpu.SemaphoreType.DMA,
            pltpu.SemaphoreType.DMA((3,)),
            pltpu.SemaphoreType.DMA((2,)),
        ],
        compiler_params=pltpu.CompilerParams(
            vmem_limit_bytes=48 << 20,
        ),
    )(x, w, b, adj_p)

    return out_p[:n, :f_out]


# speedup vs baseline: 1.0229x; 1.0229x over previous
import jax
import jax.numpy as jnp
from jax.experimental import pallas as pl
from jax.experimental.pallas import tpu as pltpu


def _round_up(x, m):
    return ((x + m - 1) // m) * m


def _make_body(n_rows, chunk):
    n_chunks = n_rows // chunk

    def _body(x_hbm, w_ref, b_ref, adj_hbm, o_hbm,
              xv, abuf, obuf, x_sem, in_sems, out_sems):
        pltpu.make_async_copy(adj_hbm.at[pl.ds(0, chunk), :],
                              abuf.at[0], in_sems.at[0]).start()
        pltpu.make_async_copy(x_hbm, xv, x_sem).start()
        if n_chunks > 1:
            pltpu.make_async_copy(adj_hbm.at[pl.ds(chunk, chunk), :],
                                  abuf.at[1], in_sems.at[1]).start()
        pltpu.make_async_copy(x_hbm, xv, x_sem).wait()

        for i in range(n_chunks):
            if i + 2 < n_chunks:
                s2 = (i + 2) % 3
                pltpu.make_async_copy(
                    adj_hbm.at[pl.ds((i + 2) * chunk, chunk), :],
                    abuf.at[s2], in_sems.at[s2]).start()
            sl = i % 3
            pltpu.make_async_copy(abuf.at[sl], abuf.at[sl],
                                  in_sems.at[sl]).wait()
            t = jnp.dot(abuf[sl], xv[...],
                        preferred_element_type=jnp.float32)
            ob = i % 2
            if i >= 2:
                pltpu.make_async_copy(obuf.at[ob], obuf.at[ob],
                                      out_sems.at[ob]).wait()
            obuf[ob] = jnp.dot(t, w_ref[...],
                               preferred_element_type=jnp.float32) + b_ref[...]
            pltpu.make_async_copy(obuf.at[ob],
                                  o_hbm.at[pl.ds(i * chunk, chunk), :],
                                  out_sems.at[ob]).start()

        for ob in range(min(2, n_chunks)):
            pltpu.make_async_copy(obuf.at[ob], obuf.at[ob],
                                  out_sems.at[ob]).wait()

    return _body


def kernel(x, adj, weight, bias):
    n, f_in = x.shape
    f_out = weight.shape[1]

    f_in_p = _round_up(f_in, 128)
    f_out_p = _round_up(f_out, 128)

    chunk = 512
    n_p = _round_up(n, chunk)

    x = x.astype(jnp.float32)
    if (n_p, f_in_p) != (n, f_in):
        x = jnp.pad(x, ((0, n_p - n), (0, f_in_p - f_in)))
    w = weight.astype(jnp.float32)
    if (f_in_p, f_out_p) != (f_in, f_out):
        w = jnp.pad(w, ((0, f_in_p - f_in), (0, f_out_p - f_out)))
    adj_p = adj if n_p == n else jnp.pad(adj, ((0, n_p - n), (0, n_p - n)))
    if bias is None:
        b = jnp.zeros((1, f_out_p), jnp.float32)
    else:
        b = jnp.pad(bias.reshape(1, f_out).astype(jnp.float32),
                    ((0, 0), (0, f_out_p - f_out)))

    out_p = pl.pallas_call(
        _make_body(n_p, chunk),
        out_shape=jax.ShapeDtypeStruct((n_p, f_out_p), jnp.float32),
        in_specs=[
            pl.BlockSpec(memory_space=pl.ANY),
            pl.BlockSpec((f_in_p, f_out_p), lambda: (0, 0)),
            pl.BlockSpec((1, f_out_p), lambda: (0, 0)),
            pl.BlockSpec(memory_space=pl.ANY),
        ],
        out_specs=pl.BlockSpec(memory_space=pl.ANY),
        scratch_shapes=[
            pltpu.VMEM((n_p, f_in_p), jnp.float32),
            pltpu.VMEM((3, chunk, n_p), jnp.float32),
            pltpu.VMEM((2, chunk, f_out_p), jnp.float32),
            pltpu.SemaphoreType.DMA,
            pltpu.SemaphoreType.DMA((3,)),
            pltpu.SemaphoreType.DMA((2,)),
        ],
        compiler_params=pltpu.CompilerParams(
            vmem_limit_bytes=48 << 20,
        ),
    )(x, w, b, adj_p)

    return out_p[:n, :f_out]
